# baseline (speedup 1.0000x reference)
import jax
import jax.numpy as jnp
from jax.experimental import pallas as pl

_B, _NFRM, _H, _W, _D = 16, 8, 24, 24, 768
_EPS = 1e-12


def _embed_ln_kernel(grid_ref, row_ref, col_ref, tt_ref, gamma_ref, beta_ref,
                     out_ref):
    g = grid_ref[0, :, 0]
    x = jnp.sum(g, axis=0) * (1.0 / _NFRM)
    row = row_ref[pl.ds(pl.program_id(1), 1)]
    x = x + row + col_ref[...] + tt_ref[...]
    mu = jnp.mean(x, axis=-1, keepdims=True)
    var = jnp.mean(jnp.square(x - mu), axis=-1, keepdims=True)
    xhat = (x - mu) * jax.lax.rsqrt(var + _EPS)
    out_ref[0] = xhat * gamma_ref[...] + beta_ref[...]


def kernel(grid, row_table, col_table, tt_table, gamma, beta):
    B, NFRM, H, W, D = grid.shape
    gamma2 = gamma.reshape(1, D)
    beta2 = beta.reshape(1, D)
    out = pl.pallas_call(
        _embed_ln_kernel,
        grid=(B, H),
        in_specs=[
            pl.BlockSpec((1, NFRM, 1, W, D), lambda b, h: (b, 0, h, 0, 0)),
            pl.BlockSpec((H, D), lambda b, h: (0, 0)),
            pl.BlockSpec((W, D), lambda b, h: (0, 0)),
            pl.BlockSpec((1, D), lambda b, h: (0, 0)),
            pl.BlockSpec((1, D), lambda b, h: (0, 0)),
            pl.BlockSpec((1, D), lambda b, h: (0, 0)),
        ],
        out_specs=pl.BlockSpec((1, W, D), lambda b, h: (b, h, 0)),
        out_shape=jax.ShapeDtypeStruct((B, H * W, D), grid.dtype),
    )(grid, row_table, col_table, tt_table, gamma2, beta2)
    return out


# speedup vs baseline: 1.8682x; 1.8682x over previous
import jax
import jax.numpy as jnp
from jax.experimental import pallas as pl
from jax.experimental.pallas import tpu as pltpu

_EPS = 1e-12


def _embed_ln_kernel(grid_ref, row_ref, col_ref, tt_ref, gamma_ref, beta_ref,
                     out_ref, acc_ref):
    f = pl.program_id(1)
    nfrm = pl.num_programs(1)
    frame = grid_ref[0, 0]

    @pl.when(f == 0)
    def _init():
        acc_ref[...] = frame

    @pl.when(f != 0)
    def _acc():
        acc_ref[...] += frame

    @pl.when(f == nfrm - 1)
    def _finish():
        x = acc_ref[...] * (1.0 / nfrm)
        x = x + row_ref[...][:, None, :] + col_ref[...][None, :, :]
        x = x + tt_ref[...][None, :, :]
        mu = jnp.mean(x, axis=-1, keepdims=True)
        var = jnp.mean(jnp.square(x - mu), axis=-1, keepdims=True)
        xhat = (x - mu) * jax.lax.rsqrt(var + _EPS)
        y = xhat * gamma_ref[...][None, :, :] + beta_ref[...][None, :, :]
        out_ref[0] = y.reshape(out_ref.shape[1], out_ref.shape[2])


def kernel(grid, row_table, col_table, tt_table, gamma, beta):
    B, NFRM, H, W, D = grid.shape
    gamma2 = gamma.reshape(1, D)
    beta2 = beta.reshape(1, D)
    out = pl.pallas_call(
        _embed_ln_kernel,
        grid=(B, NFRM),
        in_specs=[
            pl.BlockSpec((1, 1, H, W, D), lambda b, f: (b, f, 0, 0, 0)),
            pl.BlockSpec((H, D), lambda b, f: (0, 0)),
            pl.BlockSpec((W, D), lambda b, f: (0, 0)),
            pl.BlockSpec((1, D), lambda b, f: (0, 0)),
            pl.BlockSpec((1, D), lambda b, f: (0, 0)),
            pl.BlockSpec((1, D), lambda b, f: (0, 0)),
        ],
        out_specs=pl.BlockSpec((1, H * W, D), lambda b, f: (b, 0, 0)),
        out_shape=jax.ShapeDtypeStruct((B, H * W, D), grid.dtype),
        scratch_shapes=[pltpu.VMEM((H, W, D), jnp.float32)],
        compiler_params=pltpu.CompilerParams(
            dimension_semantics=("parallel", "arbitrary"),
        ),
    )(grid, row_table, col_table, tt_table, gamma2, beta2)
    return out


# speedup vs baseline: 3.4370x; 1.8397x over previous
import jax
import jax.numpy as jnp
from jax.experimental import pallas as pl
from jax.experimental.pallas import tpu as pltpu

_EPS = 1e-12


def _embed_ln_kernel(grid_ref, row_ref, col_ref, tt_ref, gamma_ref, beta_ref,
                     out_ref):
    g = grid_ref[0]
    nfrm = g.shape[0]
    x = jnp.sum(g, axis=0) * (1.0 / nfrm)
    x = x + row_ref[...][:, None, :] + col_ref[...][None, :, :]
    x = x + tt_ref[...][None, :, :]
    mu = jnp.mean(x, axis=-1, keepdims=True)
    var = jnp.mean(jnp.square(x - mu), axis=-1, keepdims=True)
    xhat = (x - mu) * jax.lax.rsqrt(var + _EPS)
    y = xhat * gamma_ref[...][None, :, :] + beta_ref[...][None, :, :]
    out_ref[0] = y.reshape(out_ref.shape[1], out_ref.shape[2])


def kernel(grid, row_table, col_table, tt_table, gamma, beta):
    B, NFRM, H, W, D = grid.shape
    gamma2 = gamma.reshape(1, D)
    beta2 = beta.reshape(1, D)
    out = pl.pallas_call(
        _embed_ln_kernel,
        grid=(B,),
        in_specs=[
            pl.BlockSpec((1, NFRM, H, W, D), lambda b: (b, 0, 0, 0, 0)),
            pl.BlockSpec((H, D), lambda b: (0, 0)),
            pl.BlockSpec((W, D), lambda b: (0, 0)),
            pl.BlockSpec((1, D), lambda b: (0, 0)),
            pl.BlockSpec((1, D), lambda b: (0, 0)),
            pl.BlockSpec((1, D), lambda b: (0, 0)),
        ],
        out_specs=pl.BlockSpec((1, H * W, D), lambda b: (b, 0, 0)),
        out_shape=jax.ShapeDtypeStruct((B, H * W, D), grid.dtype),
        compiler_params=pltpu.CompilerParams(
            dimension_semantics=("parallel",),
        ),
    )(grid, row_table, col_table, tt_table, gamma2, beta2)
    return out
